# Initial kernel scaffold; baseline (speedup 1.0000x reference)
import jax
import jax.numpy as jnp
from jax.experimental import pallas as pl


def kernel(opt_idx, opnd_type, opnd_idx, float_operand_emb, fixed_operator_emb, fixed_operand_emb, operand_table, operator_param, operand_param, ln_opt_w, ln_opt_b, ln_opnd_w, ln_opnd_b, W, b):
    raise NotImplementedError("write your pallas kernel here")


# speedup vs baseline: 9.7835x; 9.7835x over previous
import functools
import math

import jax
import jax.numpy as jnp
from jax.experimental import pallas as pl
from jax.experimental.pallas import tpu as pltpu

_D = 128
_LOG1E4 = math.log(10000.0)


def _pe_rows(indices):
    d = jnp.arange(_D)
    div = jnp.exp(((d // 2) * 2).astype(jnp.float32) * (-_LOG1E4 / _D))
    phase = indices.astype(jnp.float32)[:, None] * div[None, :]
    return jnp.where((d % 2 == 0)[None, :], jnp.sin(phase), jnp.cos(phase))


def _body(idx_ref, raw_ref, eop_ref, lnow_ref, lnob_ref, lnnw_ref, lnnb_ref,
          w_ref, bias_ref, out_ref, *, block_b):
    i = pl.program_id(0)

    raw = raw_ref[...]
    mu = jnp.mean(raw, axis=-1, keepdims=True)
    var = jnp.mean((raw - mu) ** 2, axis=-1, keepdims=True)
    lnc = (raw - mu) * jax.lax.rsqrt(var + 1e-12) * lnnw_ref[...] + lnnb_ref[...]
    ta = jax.lax.dot_general(lnc, w_ref[:, 128:256], (((1,), (1,)), ((), ())),
                             preferred_element_type=jnp.float32)
    tb = jax.lax.dot_general(lnc, w_ref[:, 256:384], (((1,), (1,)), ((), ())),
                             preferred_element_type=jnp.float32)

    idx = idx_ref[...]
    opt = idx[:, 0:1]
    c0 = idx[:, 1:2] * 16 + idx[:, 2:3]
    c1 = idx[:, 3:4] * 16 + idx[:, 4:5]

    lane16 = jax.lax.broadcasted_iota(jnp.int32, (block_b, 16), 1)
    oh_opt = (opt == lane16).astype(jnp.float32)
    e_opt = jax.lax.dot_general(oh_opt, eop_ref[...], (((1,), (0,)), ((), ())),
                                preferred_element_type=jnp.float32)

    dlane = jax.lax.broadcasted_iota(jnp.int32, (1, _D), 1)
    div = jnp.exp(((dlane // 2) * 2).astype(jnp.float32) * (-_LOG1E4 / _D))
    row = jax.lax.broadcasted_iota(jnp.int32, (block_b, 1), 0)
    pos = (row + (i * block_b + 1)).astype(jnp.float32)
    phase = pos * div
    pe = jnp.where(dlane % 2 == 0, jnp.sin(phase), jnp.cos(phase))

    x = e_opt + pe
    mu = jnp.mean(x, axis=-1, keepdims=True)
    var = jnp.mean((x - mu) ** 2, axis=-1, keepdims=True)
    lx = (x - mu) * jax.lax.rsqrt(var + 1e-12) * lnow_ref[...] + lnob_ref[...]
    out = jax.lax.dot_general(lx, w_ref[:, 0:128], (((1,), (1,)), ((), ())),
                              preferred_element_type=jnp.float32)

    lane48 = jax.lax.broadcasted_iota(jnp.int32, (block_b, 48), 1)
    oh0 = (c0 == lane48).astype(jnp.float32)
    oh1 = (c1 == lane48).astype(jnp.float32)
    out = out + jax.lax.dot_general(oh0, ta, (((1,), (0,)), ((), ())),
                                    preferred_element_type=jnp.float32)
    out = out + jax.lax.dot_general(oh1, tb, (((1,), (0,)), ((), ())),
                                    preferred_element_type=jnp.float32)
    out_ref[...] = out + bias_ref[...]


def kernel(opt_idx, opnd_type, opnd_idx, float_operand_emb, fixed_operator_emb,
           fixed_operand_emb, operand_table, operator_param, operand_param,
           ln_opt_w, ln_opt_b, ln_opnd_w, ln_opnd_b, W, b):
    n = opt_idx.shape[0]
    block_b = 1024
    nb = n // block_b

    pe16 = _pe_rows(jnp.arange(16) + 1)
    raw48 = (operand_param * jnp.repeat(operand_table, 16, axis=0)
             + jnp.concatenate([float_operand_emb[:16], pe16,
                                fixed_operand_emb[:16]], axis=0))
    eop = jnp.pad(operator_param * fixed_operator_emb,
                  ((0, 16 - fixed_operator_emb.shape[0]), (0, 0)))
    idx = jnp.stack([opt_idx, opnd_type[:, 0], opnd_idx[:, 0],
                     opnd_type[:, 1], opnd_idx[:, 1]], axis=1).astype(jnp.int32)

    full = lambda a: pl.BlockSpec(a.shape, lambda i: (0,) * a.ndim)
    lnow = ln_opt_w.reshape(1, _D)
    lnob = ln_opt_b.reshape(1, _D)
    lnnw = ln_opnd_w.reshape(1, _D)
    lnnb = ln_opnd_b.reshape(1, _D)
    bias = b.reshape(1, _D)

    return pl.pallas_call(
        functools.partial(_body, block_b=block_b),
        grid=(nb,),
        in_specs=[
            pl.BlockSpec((block_b, 5), lambda i: (i, 0)),
            full(raw48), full(eop), full(lnow), full(lnob), full(lnnw),
            full(lnnb), full(W), full(bias),
        ],
        out_specs=pl.BlockSpec((block_b, _D), lambda i: (i, 0)),
        out_shape=jax.ShapeDtypeStruct((n, _D), jnp.float32),
        compiler_params=pltpu.CompilerParams(
            dimension_semantics=("arbitrary",)),
    )(idx, raw48, eop, lnow, lnob, lnnw, lnnb, W, bias)


# speedup vs baseline: 16.9774x; 1.7353x over previous
import functools
import math

import jax
import jax.numpy as jnp
from jax.experimental import pallas as pl
from jax.experimental.pallas import tpu as pltpu

_D = 128
_LOG1E4 = math.log(10000.0)


def _pe_rows(indices):
    d = jnp.arange(_D)
    div = jnp.exp(((d // 2) * 2).astype(jnp.float32) * (-_LOG1E4 / _D))
    phase = indices.astype(jnp.float32)[:, None] * div[None, :]
    return jnp.where((d % 2 == 0)[None, :], jnp.sin(phase), jnp.cos(phase))


def _body(idx_ref, raw_ref, eop_ref, lnow_ref, lnob_ref, lnnw_ref, lnnb_ref,
          w_ref, bias_ref, out_ref, u_ref, v_ref, *, block_b):
    i = pl.program_id(0)

    dlane = jax.lax.broadcasted_iota(jnp.int32, (1, _D), 1)
    div = jnp.exp(((dlane // 2) * 2).astype(jnp.float32) * (-_LOG1E4 / _D))
    even = (dlane % 2) == 0

    @pl.when(i == 0)
    def _build_pe_tables():
        j = jax.lax.broadcasted_iota(jnp.int32, (block_b, 1), 0)
        ph = j.astype(jnp.float32) * div
        sj = jnp.sin(ph)
        cj = jnp.cos(ph)
        u_ref[...] = jnp.where(even, cj, -sj)
        v_ref[...] = jnp.where(even, sj, cj)

    raw = raw_ref[...]
    mu = jnp.mean(raw, axis=-1, keepdims=True)
    var = jnp.mean((raw - mu) ** 2, axis=-1, keepdims=True)
    lnc = (raw - mu) * jax.lax.rsqrt(var + 1e-12) * lnnw_ref[...] + lnnb_ref[...]
    tab = jnp.concatenate(
        [jax.lax.dot_general(lnc, w_ref[:, 128:256], (((1,), (1,)), ((), ())),
                             preferred_element_type=jnp.float32),
         jax.lax.dot_general(lnc, w_ref[:, 256:384], (((1,), (1,)), ((), ())),
                             preferred_element_type=jnp.float32)], axis=0)

    idx = idx_ref[...]
    opt = idx[:, 0:1]
    c0 = idx[:, 1:2] * 16 + idx[:, 2:3]
    c1 = idx[:, 3:4] * 16 + idx[:, 4:5] + 48

    lane16 = jax.lax.broadcasted_iota(jnp.int32, (block_b, 16), 1)
    oh_opt = (opt == lane16).astype(jnp.float32)
    e_opt = jax.lax.dot_general(oh_opt, eop_ref[...], (((1,), (0,)), ((), ())),
                                preferred_element_type=jnp.float32)

    base = (i * block_b + 1).astype(jnp.float32) * div
    sb = jnp.sin(base)
    cb = jnp.cos(base)
    pe = sb * u_ref[...] + cb * v_ref[...]

    x = e_opt + pe
    mu = jnp.mean(x, axis=-1, keepdims=True)
    var = jnp.mean((x - mu) ** 2, axis=-1, keepdims=True)
    lx = (x - mu) * jax.lax.rsqrt(var + 1e-12) * lnow_ref[...] + lnob_ref[...]
    out = jax.lax.dot_general(lx, w_ref[:, 0:128], (((1,), (1,)), ((), ())),
                              preferred_element_type=jnp.float32)

    lane96 = jax.lax.broadcasted_iota(jnp.int32, (block_b, 96), 1)
    oh2 = ((c0 == lane96) | (c1 == lane96)).astype(jnp.float32)
    out = out + jax.lax.dot_general(oh2, tab, (((1,), (0,)), ((), ())),
                                    preferred_element_type=jnp.float32)
    out_ref[...] = out + bias_ref[...]


def kernel(opt_idx, opnd_type, opnd_idx, float_operand_emb, fixed_operator_emb,
           fixed_operand_emb, operand_table, operator_param, operand_param,
           ln_opt_w, ln_opt_b, ln_opnd_w, ln_opnd_b, W, b):
    n = opt_idx.shape[0]
    block_b = 1024
    nb = n // block_b

    pe16 = _pe_rows(jnp.arange(16) + 1)
    raw48 = (operand_param * jnp.repeat(operand_table, 16, axis=0)
             + jnp.concatenate([float_operand_emb[:16], pe16,
                                fixed_operand_emb[:16]], axis=0))
    eop = jnp.pad(operator_param * fixed_operator_emb,
                  ((0, 16 - fixed_operator_emb.shape[0]), (0, 0)))
    idx = jnp.stack([opt_idx, opnd_type[:, 0], opnd_idx[:, 0],
                     opnd_type[:, 1], opnd_idx[:, 1]], axis=1).astype(jnp.int32)

    full = lambda a: pl.BlockSpec(a.shape, lambda i: (0,) * a.ndim)
    lnow = ln_opt_w.reshape(1, _D)
    lnob = ln_opt_b.reshape(1, _D)
    lnnw = ln_opnd_w.reshape(1, _D)
    lnnb = ln_opnd_b.reshape(1, _D)
    bias = b.reshape(1, _D)

    return pl.pallas_call(
        functools.partial(_body, block_b=block_b),
        grid=(nb,),
        in_specs=[
            pl.BlockSpec((block_b, 5), lambda i: (i, 0)),
            full(raw48), full(eop), full(lnow), full(lnob), full(lnnw),
            full(lnnb), full(W), full(bias),
        ],
        out_specs=pl.BlockSpec((block_b, _D), lambda i: (i, 0)),
        out_shape=jax.ShapeDtypeStruct((n, _D), jnp.float32),
        scratch_shapes=[pltpu.VMEM((block_b, _D), jnp.float32),
                        pltpu.VMEM((block_b, _D), jnp.float32)],
        compiler_params=pltpu.CompilerParams(
            dimension_semantics=("arbitrary",)),
    )(idx, raw48, eop, lnow, lnob, lnnw, lnnb, W, bias)
